# baseline (speedup 1.0000x reference)
import functools

import jax
import jax.numpy as jnp
from jax import lax
from jax.experimental import pallas as pl
from jax.experimental.pallas import tpu as pltpu
from jax.experimental.pallas import tpu_sc as plsc

N = 10000
F = 128
H = 16
CLS = 10
G = 64
NT = 32
C = 128
K = 80
E = 320000
EP = NT * K * C
NPAD = 10240
STRIPE = NPAD // 16
NB = 4

_mesh = plsc.VectorSubcoreMesh(core_axis_name="c", subcore_axis_name="s")


def _deg_body(dst_hbm, out_hbm, dst_v, ones_v, zb_v, deg_sh):
    c = lax.axis_index("c")
    s = lax.axis_index("s")
    wid = c * 16 + s
    for i in range(C // 16):
        ones_v[pl.ds(i * 16, 16)] = jnp.full((16,), 1.0, jnp.float32)
    for i in range(STRIPE // 16):
        zb_v[pl.ds(i * 16, 16)] = jnp.zeros((16,), jnp.float32)
    pltpu.sync_copy(zb_v, deg_sh.at[pl.ds(s * STRIPE, STRIPE)])
    pltpu.sync_copy(dst_hbm.at[wid], dst_v)
    plsc.subcore_barrier()

    def chunk(j, carry):
        pltpu.sync_copy(ones_v, deg_sh.at[dst_v.at[j]], add=True)
        return carry

    lax.fori_loop(0, K, chunk, 0)
    plsc.subcore_barrier()
    pltpu.sync_copy(deg_sh.at[pl.ds(s * STRIPE, STRIPE)],
                    out_hbm.at[c, pl.ds(s * STRIPE, STRIPE)])


_sc_params = pltpu.CompilerParams(use_tc_tiling_on_sc=False)

_deg_call = pl.kernel(
    _deg_body,
    out_type=jax.ShapeDtypeStruct((2, NPAD), jnp.float32),
    mesh=_mesh,
    compiler_params=_sc_params,
    scratch_types=[
        pltpu.VMEM((K, C), jnp.int32),
        pltpu.VMEM((C,), jnp.float32),
        pltpu.VMEM((STRIPE,), jnp.float32),
        pltpu.VMEM_SHARED((NPAD,), jnp.float32),
    ],
)


KG = 20
NG = K // KG


def _conv_body(g_hbm, src_hbm, dst_hbm, out_hbm,
               src_v, dst_v, rows0, rows1, zr_v, acc_sh,
               gsem0, gsem1, ssem0, ssem1):
    rows = (rows0, rows1)
    gsems = (gsem0, gsem1)
    ssems = (ssem0, ssem1)
    c = lax.axis_index("c")
    s = lax.axis_index("s")
    wid = c * 16 + s
    for i in range(STRIPE // NB // 2):
        zr_v[pl.ds(2 * i, 2), :] = jnp.zeros((2, 16), jnp.bfloat16)
    for q in range(NB):
        pltpu.sync_copy(
            zr_v, acc_sh.at[pl.ds(s * STRIPE + q * (STRIPE // NB), STRIPE // NB)])
    pltpu.sync_copy(src_hbm.at[wid], src_v)
    pltpu.sync_copy(dst_hbm.at[wid], dst_v)
    plsc.subcore_barrier()

    gcps = [None, None]
    scps = [[], []]
    gcps[0] = pltpu.async_copy(
        g_hbm.at[src_v.at[pl.ds(0, KG * C)]], rows[0], gsems[0])
    for t in range(NG):
        b = t % 2
        nb = (t + 1) % 2
        gcps[b].wait()
        scps[b] = [
            pltpu.async_copy(
                rows[b].at[pl.ds(j * 2 * C, 2 * C)],
                acc_sh.at[dst_v.at[t * (KG // 2) + j]], ssems[b], add=True)
            for j in range(KG // 2)
        ]
        if t + 1 < NG:
            for cp in scps[nb]:
                cp.wait()
            scps[nb] = []
            gcps[nb] = pltpu.async_copy(
                g_hbm.at[src_v.at[pl.ds((t + 1) * KG * C, KG * C)]],
                rows[nb], gsems[nb])
    for b in range(2):
        for cp in scps[b]:
            cp.wait()
    plsc.subcore_barrier()
    pltpu.sync_copy(acc_sh.at[pl.ds(s * STRIPE, STRIPE)],
                    out_hbm.at[c, pl.ds(s * STRIPE, STRIPE)])


_conv_call = pl.kernel(
    _conv_body,
    out_type=jax.ShapeDtypeStruct((2, NPAD, H), jnp.bfloat16),
    mesh=_mesh,
    compiler_params=_sc_params,
    scratch_types=[
        pltpu.VMEM((K * C,), jnp.int32),
        pltpu.VMEM((K // 2, 2 * C), jnp.int32),
        pltpu.VMEM((KG * C, H), jnp.bfloat16),
        pltpu.VMEM((KG * C, H), jnp.bfloat16),
        pltpu.VMEM((STRIPE // NB, H), jnp.bfloat16),
        pltpu.VMEM_SHARED((NPAD, H), jnp.bfloat16),
        pltpu.SemaphoreType.DMA,
        pltpu.SemaphoreType.DMA,
        pltpu.SemaphoreType.DMA,
        pltpu.SemaphoreType.DMA,
    ],
)


def _tc1_body(x_ref, w_ref, dc_ref, g_ref):
    dinv = lax.rsqrt(jnp.maximum(dc_ref[...], 1.0))
    h = jnp.dot(x_ref[...], w_ref[...], preferred_element_type=jnp.float32)
    g_ref[...] = (h * dinv).astype(jnp.bfloat16)


def _tc2_body(acc_ref, g1_ref, dc_ref, w2_ref, b1_ref, g2_ref):
    dinv = lax.rsqrt(jnp.maximum(dc_ref[...], 1.0))
    ssum = (acc_ref[0, :N, :].astype(jnp.float32)
            + acc_ref[1, :N, :].astype(jnp.float32)
            + g1_ref[...].astype(jnp.float32))
    out1 = jnp.maximum(ssum * dinv + b1_ref[...], 0.0)
    h2 = jnp.dot(out1, w2_ref[...], preferred_element_type=jnp.float32)
    g2_ref[...] = (h2 * dinv).astype(jnp.bfloat16)


def _tc3_body(acc_ref, g2_ref, dc_ref, b2_ref, batch_ref, out_ref):
    dinv = lax.rsqrt(jnp.maximum(dc_ref[...], 1.0))
    out2 = (acc_ref[0, :N, :].astype(jnp.float32)
            + acc_ref[1, :N, :].astype(jnp.float32)
            + g2_ref[...].astype(jnp.float32)) * dinv + b2_ref[...]
    gid = lax.broadcasted_iota(jnp.int32, (G, N), 0)
    ind = jnp.where(gid == batch_ref[...], 1.0, 0.0)
    seg = jnp.dot(ind, out2, preferred_element_type=jnp.float32)
    cnt = jnp.sum(ind, axis=1, keepdims=True)
    pooled = seg[:, :CLS] / jnp.maximum(cnt, 1.0)
    m = jnp.max(pooled, axis=1, keepdims=True)
    ex = pooled - m
    lse = jnp.log(jnp.sum(jnp.exp(ex), axis=1, keepdims=True))
    out_ref[...] = ex - lse


_tc1 = pl.pallas_call(
    _tc1_body, out_shape=jax.ShapeDtypeStruct((N, H), jnp.bfloat16))
_tc2 = pl.pallas_call(
    _tc2_body, out_shape=jax.ShapeDtypeStruct((N, H), jnp.bfloat16))
_tc3 = pl.pallas_call(
    _tc3_body, out_shape=jax.ShapeDtypeStruct((G, CLS), jnp.float32))


def kernel(x, edge_index, batch, W1, b1, W2, b2):
    src = edge_index[0].astype(jnp.int32)
    dst = edge_index[1].astype(jnp.int32)
    pad_per_tile = (EP - E) // NT
    srcp = jnp.concatenate(
        [src.reshape(NT, E // NT),
         jnp.zeros((NT, pad_per_tile), jnp.int32)], axis=1)
    trash = jnp.broadcast_to(
        N + jnp.arange(pad_per_tile, dtype=jnp.int32), (NT, pad_per_tile))
    dstp = jnp.concatenate([dst.reshape(NT, E // NT), trash], axis=1)
    dstp2 = dstp.reshape(NT, K // 2, 2 * C)

    degp = _deg_call(dstp.reshape(NT, K, C))
    deg_col = (degp[0, :N] + degp[1, :N] + 1.0)[:, None]

    w2p = jnp.pad(W2, ((0, 0), (0, H - CLS)))
    b1r = b1.reshape(1, H)
    b2p = jnp.pad(b2, (0, H - CLS)).reshape(1, H)
    batch2 = batch.astype(jnp.int32).reshape(1, N)

    g1 = _tc1(x, W1, deg_col)
    acc1 = _conv_call(g1, srcp, dstp2)
    g2 = _tc2(acc1, g1, deg_col, w2p, b1r)
    acc2 = _conv_call(g2, srcp, dstp2)
    return _tc3(acc2, g2, deg_col, b2p, batch2)


# speedup vs baseline: 1.0010x; 1.0010x over previous
import functools

import jax
import jax.numpy as jnp
from jax import lax
from jax.experimental import pallas as pl
from jax.experimental.pallas import tpu as pltpu
from jax.experimental.pallas import tpu_sc as plsc

N = 10000
F = 128
H = 16
CLS = 10
G = 64
NT = 32
C = 128
K = 80
E = 320000
EP = NT * K * C
NPAD = 10240
STRIPE = NPAD // 16
NB = 4

_mesh = plsc.VectorSubcoreMesh(core_axis_name="c", subcore_axis_name="s")


def _deg_body(dst_hbm, out_hbm, dst_v, ones_v, zb_v, deg_sh):
    c = lax.axis_index("c")
    s = lax.axis_index("s")
    wid = c * 16 + s
    for i in range(C // 16):
        ones_v[pl.ds(i * 16, 16)] = jnp.full((16,), 1.0, jnp.float32)
    for i in range(STRIPE // 16):
        zb_v[pl.ds(i * 16, 16)] = jnp.zeros((16,), jnp.float32)
    pltpu.sync_copy(zb_v, deg_sh.at[pl.ds(s * STRIPE, STRIPE)])
    pltpu.sync_copy(dst_hbm.at[wid], dst_v)
    plsc.subcore_barrier()

    def chunk(j, carry):
        pltpu.sync_copy(ones_v, deg_sh.at[dst_v.at[j]], add=True)
        return carry

    lax.fori_loop(0, K, chunk, 0)
    plsc.subcore_barrier()
    pltpu.sync_copy(deg_sh.at[pl.ds(s * STRIPE, STRIPE)],
                    out_hbm.at[c, pl.ds(s * STRIPE, STRIPE)])


_sc_params = pltpu.CompilerParams(use_tc_tiling_on_sc=False)

_deg_call = pl.kernel(
    _deg_body,
    out_type=jax.ShapeDtypeStruct((2, NPAD), jnp.float32),
    mesh=_mesh,
    compiler_params=_sc_params,
    scratch_types=[
        pltpu.VMEM((K, C), jnp.int32),
        pltpu.VMEM((C,), jnp.float32),
        pltpu.VMEM((STRIPE,), jnp.float32),
        pltpu.VMEM_SHARED((NPAD,), jnp.float32),
    ],
)


KG = 20
NG = K // KG


def _conv_body(g_hbm, src_hbm, dst_hbm, out_hbm,
               src_v, dst_v, rows0, rows1, zr_v, acc_sh,
               gsem0, gsem1, ssem0, ssem1):
    rows = (rows0, rows1)
    gsems = (gsem0, gsem1)
    ssems = (ssem0, ssem1)
    c = lax.axis_index("c")
    s = lax.axis_index("s")
    wid = c * 16 + s
    for i in range(STRIPE // NB // 2):
        zr_v[pl.ds(2 * i, 2), :] = jnp.zeros((2, 16), jnp.bfloat16)
    for q in range(NB):
        pltpu.sync_copy(
            zr_v, acc_sh.at[pl.ds(s * STRIPE + q * (STRIPE // NB), STRIPE // NB)])
    pltpu.sync_copy(src_hbm.at[wid], src_v)
    pltpu.sync_copy(dst_hbm.at[wid], dst_v)
    plsc.subcore_barrier()

    gcps = [None, None]
    scps = [[], []]
    gcps[0] = pltpu.async_copy(
        g_hbm.at[src_v.at[pl.ds(0, KG * C)]], rows[0], gsems[0])
    for t in range(NG):
        b = t % 2
        nb = (t + 1) % 2
        gcps[b].wait()
        scps[b] = [
            pltpu.async_copy(
                rows[b].at[pl.ds(j * 2 * C, 2 * C)],
                acc_sh.at[dst_v.at[t * (KG // 2) + j]], ssems[b], add=True)
            for j in range(KG // 2)
        ]
        if t + 1 < NG:
            for cp in scps[nb]:
                cp.wait()
            scps[nb] = []
            gcps[nb] = pltpu.async_copy(
                g_hbm.at[src_v.at[pl.ds((t + 1) * KG * C, KG * C)]],
                rows[nb], gsems[nb])
    for b in range(2):
        for cp in scps[b]:
            cp.wait()
    plsc.subcore_barrier()
    pltpu.sync_copy(acc_sh.at[pl.ds(s * STRIPE, STRIPE)],
                    out_hbm.at[c, pl.ds(s * STRIPE, STRIPE)])


_conv_call = pl.kernel(
    _conv_body,
    out_type=jax.ShapeDtypeStruct((2, NPAD, H), jnp.bfloat16),
    mesh=_mesh,
    compiler_params=_sc_params,
    scratch_types=[
        pltpu.VMEM((K * C,), jnp.int32),
        pltpu.VMEM((K // 2, 2 * C), jnp.int32),
        pltpu.VMEM((KG * C, H), jnp.bfloat16),
        pltpu.VMEM((KG * C, H), jnp.bfloat16),
        pltpu.VMEM((STRIPE // NB, H), jnp.bfloat16),
        pltpu.VMEM_SHARED((NPAD, H), jnp.bfloat16),
        pltpu.SemaphoreType.DMA,
        pltpu.SemaphoreType.DMA,
        pltpu.SemaphoreType.DMA,
        pltpu.SemaphoreType.DMA,
    ],
)


def _tc1_body(x_ref, w_ref, dc_ref, g_ref):
    dinv = lax.rsqrt(jnp.maximum(dc_ref[...], 1.0))
    h = jnp.dot(x_ref[...], w_ref[...], preferred_element_type=jnp.float32)
    g_ref[...] = (h * dinv).astype(jnp.bfloat16)


def _tc2_body(acc_ref, g1_ref, dc_ref, w2_ref, b1_ref, g2_ref):
    dinv = lax.rsqrt(jnp.maximum(dc_ref[...], 1.0))
    ssum = (acc_ref[0, :N, :].astype(jnp.float32)
            + acc_ref[1, :N, :].astype(jnp.float32)
            + g1_ref[...].astype(jnp.float32))
    out1 = jnp.maximum(ssum * dinv + b1_ref[...], 0.0)
    h2 = jnp.dot(out1, w2_ref[...], preferred_element_type=jnp.float32)
    g2_ref[...] = (h2 * dinv).astype(jnp.bfloat16)


def _tc3_body(acc_ref, g2_ref, dc_ref, b2_ref, batch_ref, out_ref):
    dinv = lax.rsqrt(jnp.maximum(dc_ref[...], 1.0))
    out2 = (acc_ref[0, :N, :].astype(jnp.float32)
            + acc_ref[1, :N, :].astype(jnp.float32)
            + g2_ref[...].astype(jnp.float32)) * dinv + b2_ref[...]
    gid = lax.broadcasted_iota(jnp.int32, (G, N), 0)
    ind = jnp.where(gid == batch_ref[...], 1.0, 0.0)
    seg = jnp.dot(ind, out2, preferred_element_type=jnp.float32)
    cnt = jnp.sum(ind, axis=1, keepdims=True)
    pooled = seg[:, :CLS] / jnp.maximum(cnt, 1.0)
    m = jnp.max(pooled, axis=1, keepdims=True)
    ex = pooled - m
    lse = jnp.log(jnp.sum(jnp.exp(ex), axis=1, keepdims=True))
    out_ref[...] = ex - lse


_tc1 = pl.pallas_call(
    _tc1_body, out_shape=jax.ShapeDtypeStruct((N, H), jnp.bfloat16))
_tc2 = pl.pallas_call(
    _tc2_body, out_shape=jax.ShapeDtypeStruct((N, H), jnp.bfloat16))
_tc3 = pl.pallas_call(
    _tc3_body, out_shape=jax.ShapeDtypeStruct((G, CLS), jnp.float32))


def kernel(x, edge_index, batch, W1, b1, W2, b2):
    src = edge_index[0].astype(jnp.int32)
    dst = edge_index[1].astype(jnp.int32)
    pad_per_tile = (EP - E) // NT
    srcp = jnp.concatenate(
        [src.reshape(NT, E // NT),
         jnp.zeros((NT, pad_per_tile), jnp.int32)], axis=1)
    trash = (N + (jnp.arange(NT, dtype=jnp.int32)[:, None] % 16) * 15
             + (jnp.arange(pad_per_tile, dtype=jnp.int32)[None, :] % 15))
    dstp = jnp.concatenate([dst.reshape(NT, E // NT), trash], axis=1)
    dstp2 = dstp.reshape(NT, K // 2, 2 * C)

    degp = _deg_call(dstp.reshape(NT, K, C))
    deg_col = (degp[0, :N] + degp[1, :N] + 1.0)[:, None]

    w2p = jnp.pad(W2, ((0, 0), (0, H - CLS)))
    b1r = b1.reshape(1, H)
    b2p = jnp.pad(b2, (0, H - CLS)).reshape(1, H)
    batch2 = batch.astype(jnp.int32).reshape(1, N)

    g1 = _tc1(x, W1, deg_col)
    acc1 = _conv_call(g1, srcp, dstp2)
    g2 = _tc2(acc1, g1, deg_col, w2p, b1r)
    acc2 = _conv_call(g2, srcp, dstp2)
    return _tc3(acc2, g2, deg_col, b2p, batch2)


# speedup vs baseline: 1.4697x; 1.4682x over previous
import functools

import jax
import jax.numpy as jnp
from jax import lax
from jax.experimental import pallas as pl
from jax.experimental.pallas import tpu as pltpu
from jax.experimental.pallas import tpu_sc as plsc

N = 10000
F = 128
H = 16
CLS = 10
G = 64
NT = 32
C = 128
E = 320000
NCH = E // C
CH = NCH // NT
LEFT = NCH - CH * NT
NPAD = 10240
STRIPE = NPAD // 16
NB = 4
KG = 26
NG = CH // KG

_mesh = plsc.VectorSubcoreMesh(core_axis_name="c", subcore_axis_name="s")


def _deg_body(dst_hbm, out_hbm, dst_v, ones_v, zb_v, deg_sh):
    c = lax.axis_index("c")
    s = lax.axis_index("s")
    wid = c * 16 + s
    for i in range(C // 16):
        ones_v[pl.ds(i * 16, 16)] = jnp.full((16,), 1.0, jnp.float32)
    for i in range(STRIPE // 16):
        zb_v[pl.ds(i * 16, 16)] = jnp.zeros((16,), jnp.float32)
    pltpu.sync_copy(zb_v, deg_sh.at[pl.ds(s * STRIPE, STRIPE)])
    pltpu.sync_copy(dst_hbm.at[pl.ds(wid * CH, CH)], dst_v.at[pl.ds(0, CH)])

    @pl.when(wid < LEFT)
    def _load_left():
        pltpu.sync_copy(dst_hbm.at[pl.ds(NT * CH + wid, 1)],
                        dst_v.at[pl.ds(CH, 1)])

    plsc.subcore_barrier()

    def chunk(j, carry):
        pltpu.sync_copy(ones_v, deg_sh.at[dst_v.at[j]], add=True)
        return carry

    lax.fori_loop(0, CH, chunk, 0)

    @pl.when(wid < LEFT)
    def _scatter_left():
        pltpu.sync_copy(ones_v, deg_sh.at[dst_v.at[CH]], add=True)

    plsc.subcore_barrier()
    pltpu.sync_copy(deg_sh.at[pl.ds(s * STRIPE, STRIPE)],
                    out_hbm.at[c, pl.ds(s * STRIPE, STRIPE)])


_sc_params = pltpu.CompilerParams(use_tc_tiling_on_sc=False)

_deg_call = pl.kernel(
    _deg_body,
    out_type=jax.ShapeDtypeStruct((2, NPAD), jnp.float32),
    mesh=_mesh,
    compiler_params=_sc_params,
    scratch_types=[
        pltpu.VMEM((CH + 1, C), jnp.int32),
        pltpu.VMEM((C,), jnp.float32),
        pltpu.VMEM((STRIPE,), jnp.float32),
        pltpu.VMEM_SHARED((NPAD,), jnp.float32),
    ],
)


def _conv_body(g_hbm, src_hbm, dst_hbm, out_hbm,
               src_v, dst_v, rows0, rows1, rows_l, zr_v, acc_sh,
               gsem0, gsem1, ssem0, ssem1):
    rows = (rows0, rows1)
    gsems = (gsem0, gsem1)
    ssems = (ssem0, ssem1)
    c = lax.axis_index("c")
    s = lax.axis_index("s")
    wid = c * 16 + s
    for i in range(STRIPE // NB // 2):
        zr_v[pl.ds(2 * i, 2), :] = jnp.zeros((2, 16), jnp.bfloat16)
    for q in range(NB):
        pltpu.sync_copy(
            zr_v, acc_sh.at[pl.ds(s * STRIPE + q * (STRIPE // NB), STRIPE // NB)])
    pltpu.sync_copy(src_hbm.at[pl.ds(wid * CH * C, CH * C)],
                    src_v.at[pl.ds(0, CH * C)])
    pltpu.sync_copy(dst_hbm.at[pl.ds(wid * CH, CH)], dst_v.at[pl.ds(0, CH)])

    @pl.when(wid < LEFT)
    def _load_left():
        pltpu.sync_copy(src_hbm.at[pl.ds((NT * CH + wid) * C, C)],
                        src_v.at[pl.ds(CH * C, C)])
        pltpu.sync_copy(dst_hbm.at[pl.ds(NT * CH + wid, 1)],
                        dst_v.at[pl.ds(CH, 1)])

    plsc.subcore_barrier()

    gcps = [None, None]
    scps = [[], []]
    gcps[0] = pltpu.async_copy(
        g_hbm.at[src_v.at[pl.ds(0, KG * C)]], rows[0], gsems[0])
    for t in range(NG):
        b = t % 2
        nb = (t + 1) % 2
        gcps[b].wait()
        scps[b] = [
            pltpu.async_copy(
                rows[b].at[pl.ds(j * C, C)],
                acc_sh.at[dst_v.at[t * KG + j]], ssems[b], add=True)
            for j in range(KG)
        ]
        if t + 1 < NG:
            for cp in scps[nb]:
                cp.wait()
            scps[nb] = []
            gcps[nb] = pltpu.async_copy(
                g_hbm.at[src_v.at[pl.ds((t + 1) * KG * C, KG * C)]],
                rows[nb], gsems[nb])
    for b in range(2):
        for cp in scps[b]:
            cp.wait()

    @pl.when(wid < LEFT)
    def _do_left():
        pltpu.async_copy(
            g_hbm.at[src_v.at[pl.ds(CH * C, C)]], rows_l, gsems[0]).wait()
        pltpu.sync_copy(rows_l, acc_sh.at[dst_v.at[CH]], add=True)

    plsc.subcore_barrier()
    pltpu.sync_copy(acc_sh.at[pl.ds(s * STRIPE, STRIPE)],
                    out_hbm.at[c, pl.ds(s * STRIPE, STRIPE)])


_conv_call = pl.kernel(
    _conv_body,
    out_type=jax.ShapeDtypeStruct((2, NPAD, H), jnp.bfloat16),
    mesh=_mesh,
    compiler_params=_sc_params,
    scratch_types=[
        pltpu.VMEM(((CH + 1) * C,), jnp.int32),
        pltpu.VMEM((CH + 1, C), jnp.int32),
        pltpu.VMEM((KG * C, H), jnp.bfloat16),
        pltpu.VMEM((KG * C, H), jnp.bfloat16),
        pltpu.VMEM((C, H), jnp.bfloat16),
        pltpu.VMEM((STRIPE // NB, H), jnp.bfloat16),
        pltpu.VMEM_SHARED((NPAD, H), jnp.bfloat16),
        pltpu.SemaphoreType.DMA,
        pltpu.SemaphoreType.DMA,
        pltpu.SemaphoreType.DMA,
        pltpu.SemaphoreType.DMA,
    ],
)


def _tc1_body(x_ref, w_ref, dc_ref, g_ref):
    dinv = lax.rsqrt(jnp.maximum(dc_ref[...], 1.0))
    h = jnp.dot(x_ref[...], w_ref[...], preferred_element_type=jnp.float32)
    g_ref[...] = (h * dinv).astype(jnp.bfloat16)


def _tc2_body(acc_ref, g1_ref, dc_ref, w2_ref, b1_ref, g2_ref):
    dinv = lax.rsqrt(jnp.maximum(dc_ref[...], 1.0))
    ssum = (acc_ref[0, :N, :].astype(jnp.float32)
            + acc_ref[1, :N, :].astype(jnp.float32)
            + g1_ref[...].astype(jnp.float32))
    out1 = jnp.maximum(ssum * dinv + b1_ref[...], 0.0)
    h2 = jnp.dot(out1, w2_ref[...], preferred_element_type=jnp.float32)
    g2_ref[...] = (h2 * dinv).astype(jnp.bfloat16)


def _tc3_body(acc_ref, g2_ref, dc_ref, b2_ref, batch_ref, out_ref):
    dinv = lax.rsqrt(jnp.maximum(dc_ref[...], 1.0))
    out2 = (acc_ref[0, :N, :].astype(jnp.float32)
            + acc_ref[1, :N, :].astype(jnp.float32)
            + g2_ref[...].astype(jnp.float32)) * dinv + b2_ref[...]
    gid = lax.broadcasted_iota(jnp.int32, (G, N), 0)
    ind = jnp.where(gid == batch_ref[...], 1.0, 0.0)
    seg = jnp.dot(ind, out2, preferred_element_type=jnp.float32)
    cnt = jnp.sum(ind, axis=1, keepdims=True)
    pooled = seg[:, :CLS] / jnp.maximum(cnt, 1.0)
    m = jnp.max(pooled, axis=1, keepdims=True)
    ex = pooled - m
    lse = jnp.log(jnp.sum(jnp.exp(ex), axis=1, keepdims=True))
    out_ref[...] = ex - lse


_tc1 = pl.pallas_call(
    _tc1_body, out_shape=jax.ShapeDtypeStruct((N, H), jnp.bfloat16))
_tc2 = pl.pallas_call(
    _tc2_body, out_shape=jax.ShapeDtypeStruct((N, H), jnp.bfloat16))
_tc3 = pl.pallas_call(
    _tc3_body, out_shape=jax.ShapeDtypeStruct((G, CLS), jnp.float32))


def kernel(x, edge_index, batch, W1, b1, W2, b2):
    src = edge_index[0].astype(jnp.int32)
    dst2 = edge_index[1].astype(jnp.int32).reshape(NCH, C)

    degp = _deg_call(dst2)
    deg_col = (degp[0, :N] + degp[1, :N] + 1.0)[:, None]

    w2p = jnp.pad(W2, ((0, 0), (0, H - CLS)))
    b1r = b1.reshape(1, H)
    b2p = jnp.pad(b2, (0, H - CLS)).reshape(1, H)
    batch2 = batch.astype(jnp.int32).reshape(1, N)

    g1 = _tc1(x, W1, deg_col)
    acc1 = _conv_call(g1, src, dst2)
    g2 = _tc2(acc1, g1, deg_col, w2p, b1r)
    acc2 = _conv_call(g2, src, dst2)
    return _tc3(acc2, g2, deg_col, b2p, batch2)


# speedup vs baseline: 1.5040x; 1.0233x over previous
import functools

import jax
import jax.numpy as jnp
from jax import lax
from jax.experimental import pallas as pl
from jax.experimental.pallas import tpu as pltpu
from jax.experimental.pallas import tpu_sc as plsc

N = 10000
F = 128
H = 16
CLS = 10
G = 64
NT = 32
C = 128
E = 320000
NCH = E // C
CH = NCH // NT
LEFT = NCH - CH * NT
NPAD = 10240
STRIPE = NPAD // 16
NB = 4
KG = 26
NG = CH // KG

_mesh = plsc.VectorSubcoreMesh(core_axis_name="c", subcore_axis_name="s")


def _deg_body(dst_hbm, out_hbm, dst_v, ones_v, zb_v, deg_sh):
    c = lax.axis_index("c")
    s = lax.axis_index("s")
    wid = c * 16 + s
    for i in range(C // 16):
        ones_v[pl.ds(i * 16, 16)] = jnp.full((16,), 1.0, jnp.float32)
    for i in range(STRIPE // 16):
        zb_v[pl.ds(i * 16, 16)] = jnp.zeros((16,), jnp.float32)
    pltpu.sync_copy(zb_v, deg_sh.at[pl.ds(s * STRIPE, STRIPE)])
    pltpu.sync_copy(dst_hbm.at[pl.ds(wid * CH, CH)], dst_v.at[pl.ds(0, CH)])

    @pl.when(wid < LEFT)
    def _load_left():
        pltpu.sync_copy(dst_hbm.at[pl.ds(NT * CH + wid, 1)],
                        dst_v.at[pl.ds(CH, 1)])

    plsc.subcore_barrier()

    def chunk(j, carry):
        pltpu.sync_copy(ones_v, deg_sh.at[dst_v.at[j]], add=True)
        return carry

    lax.fori_loop(0, CH, chunk, 0)

    @pl.when(wid < LEFT)
    def _scatter_left():
        pltpu.sync_copy(ones_v, deg_sh.at[dst_v.at[CH]], add=True)

    plsc.subcore_barrier()
    pltpu.sync_copy(deg_sh.at[pl.ds(s * STRIPE, STRIPE)],
                    out_hbm.at[c, pl.ds(s * STRIPE, STRIPE)])


_sc_params = pltpu.CompilerParams(use_tc_tiling_on_sc=False)

_deg_call = pl.kernel(
    _deg_body,
    out_type=jax.ShapeDtypeStruct((2, NPAD), jnp.float32),
    mesh=_mesh,
    compiler_params=_sc_params,
    scratch_types=[
        pltpu.VMEM((CH + 1, C), jnp.int32),
        pltpu.VMEM((C,), jnp.float32),
        pltpu.VMEM((STRIPE,), jnp.float32),
        pltpu.VMEM_SHARED((NPAD,), jnp.float32),
    ],
)


def _conv_body(g_hbm, src_hbm, dst_hbm, out_hbm,
               src_v, dst_v, rows0, rows1, rows_l, zr_v, acc_sh,
               gsem0, gsem1, ssem0, ssem1):
    rows = (rows0, rows1)
    gsems = (gsem0, gsem1)
    ssems = (ssem0, ssem1)
    c = lax.axis_index("c")
    s = lax.axis_index("s")
    wid = c * 16 + s
    for i in range(STRIPE // NB // 2):
        zr_v[pl.ds(2 * i, 2), :] = jnp.zeros((2, 16), jnp.bfloat16)
    for q in range(NB):
        pltpu.sync_copy(
            zr_v, acc_sh.at[pl.ds(s * STRIPE + q * (STRIPE // NB), STRIPE // NB)])
    pltpu.sync_copy(src_hbm.at[pl.ds(wid * CH * C, CH * C)],
                    src_v.at[pl.ds(0, CH * C)])
    pltpu.sync_copy(dst_hbm.at[pl.ds(wid * CH, CH)], dst_v.at[pl.ds(0, CH)])

    @pl.when(wid < LEFT)
    def _load_left():
        pltpu.sync_copy(src_hbm.at[pl.ds((NT * CH + wid) * C, C)],
                        src_v.at[pl.ds(CH * C, C)])
        pltpu.sync_copy(dst_hbm.at[pl.ds(NT * CH + wid, 1)],
                        dst_v.at[pl.ds(CH, 1)])

    plsc.subcore_barrier()

    gcps = [None, None]
    scps = [[], []]
    gcps[0] = pltpu.async_copy(
        g_hbm.at[src_v.at[pl.ds(0, KG * C)]], rows[0], gsems[0])
    for t in range(NG):
        b = t % 2
        nb = (t + 1) % 2
        gcps[b].wait()
        scps[b] = [
            pltpu.async_copy(
                rows[b].at[pl.ds(j * C, C)],
                acc_sh.at[dst_v.at[t * KG + j]], ssems[b], add=True)
            for j in range(KG)
        ]
        if t + 1 < NG:
            for cp in scps[nb]:
                cp.wait()
            scps[nb] = []
            gcps[nb] = pltpu.async_copy(
                g_hbm.at[src_v.at[pl.ds((t + 1) * KG * C, KG * C)]],
                rows[nb], gsems[nb])
    for b in range(2):
        for cp in scps[b]:
            cp.wait()

    @pl.when(wid < LEFT)
    def _do_left():
        pltpu.async_copy(
            g_hbm.at[src_v.at[pl.ds(CH * C, C)]], rows_l, gsems[0]).wait()
        pltpu.sync_copy(rows_l, acc_sh.at[dst_v.at[CH]], add=True)

    plsc.subcore_barrier()
    pltpu.sync_copy(acc_sh.at[pl.ds(s * STRIPE, STRIPE)],
                    out_hbm.at[c, pl.ds(s * STRIPE, STRIPE)])


_conv_call = pl.kernel(
    _conv_body,
    out_type=jax.ShapeDtypeStruct((2, NPAD, H), jnp.bfloat16),
    mesh=_mesh,
    compiler_params=_sc_params,
    scratch_types=[
        pltpu.VMEM(((CH + 1) * C,), jnp.int32),
        pltpu.VMEM((CH + 1, C), jnp.int32),
        pltpu.VMEM((KG * C, H), jnp.bfloat16),
        pltpu.VMEM((KG * C, H), jnp.bfloat16),
        pltpu.VMEM((C, H), jnp.bfloat16),
        pltpu.VMEM((STRIPE // NB, H), jnp.bfloat16),
        pltpu.VMEM_SHARED((NPAD, H), jnp.bfloat16),
        pltpu.SemaphoreType.DMA,
        pltpu.SemaphoreType.DMA,
        pltpu.SemaphoreType.DMA,
        pltpu.SemaphoreType.DMA,
    ],
)


def _tc1_body(x_ref, w_ref, degp_ref, g_ref, dinv_ref):
    deg = degp_ref[0, :] + degp_ref[1, :] + 1.0
    dinv = lax.rsqrt(jnp.maximum(deg, 1.0))[:N][:, None]
    h = jnp.dot(x_ref[...], w_ref[...], preferred_element_type=jnp.float32)
    g_ref[...] = (h * dinv).astype(jnp.bfloat16)
    dinv_ref[...] = dinv


def _tc2_body(acc_ref, g1_ref, dc_ref, w2_ref, b1_ref, g2_ref):
    dinv = dc_ref[...]
    ssum = (acc_ref[0, :N, :].astype(jnp.float32)
            + acc_ref[1, :N, :].astype(jnp.float32)
            + g1_ref[...].astype(jnp.float32))
    out1 = jnp.maximum(ssum * dinv + b1_ref[...], 0.0)
    h2 = jnp.dot(out1, w2_ref[...], preferred_element_type=jnp.float32)
    g2_ref[...] = (h2 * dinv).astype(jnp.bfloat16)


def _tc3_body(acc_ref, g2_ref, dc_ref, b2_ref, batch_ref, out_ref):
    dinv = dc_ref[...]
    out2 = (acc_ref[0, :N, :].astype(jnp.float32)
            + acc_ref[1, :N, :].astype(jnp.float32)
            + g2_ref[...].astype(jnp.float32)) * dinv + b2_ref[...]
    gid = lax.broadcasted_iota(jnp.int32, (G, N), 0)
    ind = jnp.where(gid == batch_ref[...], 1.0, 0.0)
    seg = jnp.dot(ind, out2, preferred_element_type=jnp.float32)
    cnt = jnp.sum(ind, axis=1, keepdims=True)
    pooled = seg[:, :CLS] / jnp.maximum(cnt, 1.0)
    m = jnp.max(pooled, axis=1, keepdims=True)
    ex = pooled - m
    lse = jnp.log(jnp.sum(jnp.exp(ex), axis=1, keepdims=True))
    out_ref[...] = ex - lse


_tc1 = pl.pallas_call(
    _tc1_body, out_shape=(jax.ShapeDtypeStruct((N, H), jnp.bfloat16),
                          jax.ShapeDtypeStruct((N, 1), jnp.float32)))
_tc2 = pl.pallas_call(
    _tc2_body, out_shape=jax.ShapeDtypeStruct((N, H), jnp.bfloat16))
_tc3 = pl.pallas_call(
    _tc3_body, out_shape=jax.ShapeDtypeStruct((G, CLS), jnp.float32))


def kernel(x, edge_index, batch, W1, b1, W2, b2):
    src = edge_index[0].astype(jnp.int32)
    dst2 = edge_index[1].astype(jnp.int32).reshape(NCH, C)

    degp = _deg_call(dst2)

    w2p = jnp.pad(W2, ((0, 0), (0, H - CLS)))
    b1r = b1.reshape(1, H)
    b2p = jnp.pad(b2, (0, H - CLS)).reshape(1, H)
    batch2 = batch.astype(jnp.int32).reshape(1, N)

    g1, dinv_col = _tc1(x, W1, degp)
    acc1 = _conv_call(g1, src, dst2)
    g2 = _tc2(acc1, g1, dinv_col, w2p, b1r)
    acc2 = _conv_call(g2, src, dst2)
    return _tc3(acc2, g2, dinv_col, b2p, batch2)


# speedup vs baseline: 1.5066x; 1.0018x over previous
import functools

import jax
import jax.numpy as jnp
from jax import lax
from jax.experimental import pallas as pl
from jax.experimental.pallas import tpu as pltpu
from jax.experimental.pallas import tpu_sc as plsc

N = 10000
F = 128
H = 16
CLS = 10
G = 64
NT = 32
C = 128
E = 320000
NCH = E // C
CH = NCH // NT
LEFT = NCH - CH * NT
NPAD = 10240
STRIPE = NPAD // 16
NB = 4
KG = 26
NG = CH // KG

_mesh = plsc.VectorSubcoreMesh(core_axis_name="c", subcore_axis_name="s")


def _deg_body(dst_hbm, out_hbm, dst_v, ones_v, zb_v, deg_sh):
    c = lax.axis_index("c")
    s = lax.axis_index("s")
    wid = c * 16 + s
    for i in range(C // 16):
        ones_v[pl.ds(i * 16, 16)] = jnp.full((16,), 1.0, jnp.float32)
    for i in range(STRIPE // 16):
        zb_v[pl.ds(i * 16, 16)] = jnp.zeros((16,), jnp.float32)
    pltpu.sync_copy(zb_v, deg_sh.at[pl.ds(s * STRIPE, STRIPE)])
    pltpu.sync_copy(dst_hbm.at[pl.ds(wid * CH, CH)], dst_v.at[pl.ds(0, CH)])

    @pl.when(wid < LEFT)
    def _load_left():
        pltpu.sync_copy(dst_hbm.at[pl.ds(NT * CH + wid, 1)],
                        dst_v.at[pl.ds(CH, 1)])

    plsc.subcore_barrier()

    def chunk(j, carry):
        pltpu.sync_copy(ones_v, deg_sh.at[dst_v.at[j]], add=True)
        return carry

    lax.fori_loop(0, CH, chunk, 0)

    @pl.when(wid < LEFT)
    def _scatter_left():
        pltpu.sync_copy(ones_v, deg_sh.at[dst_v.at[CH]], add=True)

    plsc.subcore_barrier()
    pltpu.sync_copy(deg_sh.at[pl.ds(s * STRIPE, STRIPE)],
                    out_hbm.at[c, pl.ds(s * STRIPE, STRIPE)])


_sc_params = pltpu.CompilerParams(use_tc_tiling_on_sc=False)

_deg_call = pl.kernel(
    _deg_body,
    out_type=jax.ShapeDtypeStruct((2, NPAD), jnp.float32),
    mesh=_mesh,
    compiler_params=_sc_params,
    scratch_types=[
        pltpu.VMEM((CH + 1, C), jnp.int32),
        pltpu.VMEM((C,), jnp.float32),
        pltpu.VMEM((STRIPE,), jnp.float32),
        pltpu.VMEM_SHARED((NPAD,), jnp.float32),
    ],
)


def _conv_body(g_hbm, src_hbm, dst_hbm, out_hbm,
               src_v, dst_v, rows0, rows1, rows_l, zr_v, acc_sh,
               gsem0, gsem1, ssem0, ssem1):
    rows = (rows0, rows1)
    gsems = (gsem0, gsem1)
    ssems = (ssem0, ssem1)
    c = lax.axis_index("c")
    s = lax.axis_index("s")
    wid = c * 16 + s
    for i in range(STRIPE // NB // 2):
        zr_v[pl.ds(2 * i, 2), :] = jnp.zeros((2, 16), jnp.bfloat16)
    for q in range(NB):
        pltpu.sync_copy(
            zr_v, acc_sh.at[pl.ds(s * STRIPE + q * (STRIPE // NB), STRIPE // NB)])
    pltpu.sync_copy(src_hbm.at[pl.ds(wid * CH * C, CH * C)],
                    src_v.at[pl.ds(0, CH * C)])
    pltpu.sync_copy(dst_hbm.at[pl.ds(wid * CH, CH)], dst_v.at[pl.ds(0, CH)])

    @pl.when(wid < LEFT)
    def _load_left():
        pltpu.sync_copy(src_hbm.at[pl.ds((NT * CH + wid) * C, C)],
                        src_v.at[pl.ds(CH * C, C)])
        pltpu.sync_copy(dst_hbm.at[pl.ds(NT * CH + wid, 1)],
                        dst_v.at[pl.ds(CH, 1)])

    plsc.subcore_barrier()

    def _fire(t, b):
        def body(j, carry):
            pltpu.async_copy(
                rows[b].at[pl.ds(j * C, C)],
                acc_sh.at[dst_v.at[t * KG + j]], ssems[b], add=True)
            return carry
        lax.fori_loop(0, KG, body, 0)

    def _drain(t, b):
        def body(j, carry):
            pltpu.make_async_copy(
                rows[b].at[pl.ds(j * C, C)],
                acc_sh.at[dst_v.at[t * KG + j]], ssems[b]).wait()
            return carry
        lax.fori_loop(0, KG, body, 0)

    gcps = [None, None]
    gcps[0] = pltpu.async_copy(
        g_hbm.at[src_v.at[pl.ds(0, KG * C)]], rows[0], gsems[0])
    for t in range(NG):
        b = t % 2
        nb = (t + 1) % 2
        gcps[b].wait()
        _fire(t, b)
        if t + 1 < NG:
            if t > 0:
                _drain(t - 1, nb)
            gcps[nb] = pltpu.async_copy(
                g_hbm.at[src_v.at[pl.ds((t + 1) * KG * C, KG * C)]],
                rows[nb], gsems[nb])
    _drain(NG - 2, 1)
    _drain(NG - 1, 0)

    @pl.when(wid < LEFT)
    def _do_left():
        pltpu.async_copy(
            g_hbm.at[src_v.at[pl.ds(CH * C, C)]], rows_l, gsems[0]).wait()
        pltpu.sync_copy(rows_l, acc_sh.at[dst_v.at[CH]], add=True)

    plsc.subcore_barrier()
    pltpu.sync_copy(acc_sh.at[pl.ds(s * STRIPE, STRIPE)],
                    out_hbm.at[c, pl.ds(s * STRIPE, STRIPE)])


_conv_call = pl.kernel(
    _conv_body,
    out_type=jax.ShapeDtypeStruct((2, NPAD, H), jnp.bfloat16),
    mesh=_mesh,
    compiler_params=_sc_params,
    scratch_types=[
        pltpu.VMEM(((CH + 1) * C,), jnp.int32),
        pltpu.VMEM((CH + 1, C), jnp.int32),
        pltpu.VMEM((KG * C, H), jnp.bfloat16),
        pltpu.VMEM((KG * C, H), jnp.bfloat16),
        pltpu.VMEM((C, H), jnp.bfloat16),
        pltpu.VMEM((STRIPE // NB, H), jnp.bfloat16),
        pltpu.VMEM_SHARED((NPAD, H), jnp.bfloat16),
        pltpu.SemaphoreType.DMA,
        pltpu.SemaphoreType.DMA,
        pltpu.SemaphoreType.DMA,
        pltpu.SemaphoreType.DMA,
    ],
)


def _tc1_body(x_ref, w_ref, degp_ref, g_ref, dinv_ref):
    deg = degp_ref[0, :] + degp_ref[1, :] + 1.0
    dinv = lax.rsqrt(jnp.maximum(deg, 1.0))[:N][:, None]
    h = jnp.dot(x_ref[...], w_ref[...], preferred_element_type=jnp.float32)
    g_ref[...] = (h * dinv).astype(jnp.bfloat16)
    dinv_ref[...] = dinv


def _tc2_body(acc_ref, g1_ref, dc_ref, w2_ref, b1_ref, g2_ref):
    dinv = dc_ref[...]
    ssum = (acc_ref[0, :N, :].astype(jnp.float32)
            + acc_ref[1, :N, :].astype(jnp.float32)
            + g1_ref[...].astype(jnp.float32))
    out1 = jnp.maximum(ssum * dinv + b1_ref[...], 0.0)
    h2 = jnp.dot(out1, w2_ref[...], preferred_element_type=jnp.float32)
    g2_ref[...] = (h2 * dinv).astype(jnp.bfloat16)


def _tc3_body(acc_ref, g2_ref, dc_ref, b2_ref, batch_ref, out_ref):
    dinv = dc_ref[...]
    out2 = (acc_ref[0, :N, :].astype(jnp.float32)
            + acc_ref[1, :N, :].astype(jnp.float32)
            + g2_ref[...].astype(jnp.float32)) * dinv + b2_ref[...]
    gid = lax.broadcasted_iota(jnp.int32, (G, N), 0)
    ind = jnp.where(gid == batch_ref[...], 1.0, 0.0)
    seg = jnp.dot(ind, out2, preferred_element_type=jnp.float32)
    cnt = jnp.sum(ind, axis=1, keepdims=True)
    pooled = seg[:, :CLS] / jnp.maximum(cnt, 1.0)
    m = jnp.max(pooled, axis=1, keepdims=True)
    ex = pooled - m
    lse = jnp.log(jnp.sum(jnp.exp(ex), axis=1, keepdims=True))
    out_ref[...] = ex - lse


_tc1 = pl.pallas_call(
    _tc1_body, out_shape=(jax.ShapeDtypeStruct((N, H), jnp.bfloat16),
                          jax.ShapeDtypeStruct((N, 1), jnp.float32)))
_tc2 = pl.pallas_call(
    _tc2_body, out_shape=jax.ShapeDtypeStruct((N, H), jnp.bfloat16))
_tc3 = pl.pallas_call(
    _tc3_body, out_shape=jax.ShapeDtypeStruct((G, CLS), jnp.float32))


def kernel(x, edge_index, batch, W1, b1, W2, b2):
    src = edge_index[0].astype(jnp.int32)
    dst2 = edge_index[1].astype(jnp.int32).reshape(NCH, C)

    degp = _deg_call(dst2)

    w2p = jnp.pad(W2, ((0, 0), (0, H - CLS)))
    b1r = b1.reshape(1, H)
    b2p = jnp.pad(b2, (0, H - CLS)).reshape(1, H)
    batch2 = batch.astype(jnp.int32).reshape(1, N)

    g1, dinv_col = _tc1(x, W1, degp)
    acc1 = _conv_call(g1, src, dst2)
    g2 = _tc2(acc1, g1, dinv_col, w2p, b1r)
    acc2 = _conv_call(g2, src, dst2)
    return _tc3(acc2, g2, dinv_col, b2p, batch2)


# speedup vs baseline: 1.5632x; 1.0376x over previous
import functools

import jax
import jax.numpy as jnp
from jax import lax
from jax.experimental import pallas as pl
from jax.experimental.pallas import tpu as pltpu
from jax.experimental.pallas import tpu_sc as plsc

N = 10000
F = 128
H = 16
CLS = 10
G = 64
NT = 32
C = 128
E = 320000
NCH = E // C
CH = NCH // NT
LEFT = NCH - CH * NT
NPAD = 10240
STRIPE = NPAD // 16
NB = 4
KG = 26
NG = CH // KG

_mesh = plsc.VectorSubcoreMesh(core_axis_name="c", subcore_axis_name="s")


def _deg_body(dst_hbm, out_hbm, dst_v, ones_v, zb_v, deg_sh, dsem):
    c = lax.axis_index("c")
    s = lax.axis_index("s")
    wid = c * 16 + s
    for i in range(C // 16):
        ones_v[pl.ds(i * 16, 16)] = jnp.full((16,), 1.0, jnp.float32)
    for i in range(STRIPE // 16):
        zb_v[pl.ds(i * 16, 16)] = jnp.zeros((16,), jnp.float32)
    pltpu.sync_copy(zb_v, deg_sh.at[pl.ds(s * STRIPE, STRIPE)])
    pltpu.sync_copy(dst_hbm.at[pl.ds(wid * CH, CH)], dst_v.at[pl.ds(0, CH)])

    @pl.when(wid < LEFT)
    def _load_left():
        pltpu.sync_copy(dst_hbm.at[pl.ds(NT * CH + wid, 1)],
                        dst_v.at[pl.ds(CH, 1)])

    plsc.subcore_barrier()

    def chunk(j, carry):
        pltpu.async_copy(ones_v, deg_sh.at[dst_v.at[j]], dsem, add=True)
        return carry

    lax.fori_loop(0, CH, chunk, 0)

    def chunk_drain(j, carry):
        pltpu.make_async_copy(ones_v, deg_sh.at[dst_v.at[j]], dsem).wait()
        return carry

    lax.fori_loop(0, CH, chunk_drain, 0)

    @pl.when(wid < LEFT)
    def _scatter_left():
        pltpu.sync_copy(ones_v, deg_sh.at[dst_v.at[CH]], add=True)

    plsc.subcore_barrier()
    pltpu.sync_copy(deg_sh.at[pl.ds(s * STRIPE, STRIPE)],
                    out_hbm.at[c, pl.ds(s * STRIPE, STRIPE)])


_sc_params = pltpu.CompilerParams(use_tc_tiling_on_sc=False)

_deg_call = pl.kernel(
    _deg_body,
    out_type=jax.ShapeDtypeStruct((2, NPAD), jnp.float32),
    mesh=_mesh,
    compiler_params=_sc_params,
    scratch_types=[
        pltpu.VMEM((CH + 1, C), jnp.int32),
        pltpu.VMEM((C,), jnp.float32),
        pltpu.VMEM((STRIPE,), jnp.float32),
        pltpu.VMEM_SHARED((NPAD,), jnp.float32),
        pltpu.SemaphoreType.DMA,
    ],
)


def _conv_body(g_hbm, src_hbm, dst_hbm, out_hbm,
               src_v, dst_v, rows0, rows1, rows_l, zr_v, acc_sh,
               gsem0, gsem1, ssem0, ssem1):
    rows = (rows0, rows1)
    gsems = (gsem0, gsem1)
    ssems = (ssem0, ssem1)
    c = lax.axis_index("c")
    s = lax.axis_index("s")
    wid = c * 16 + s
    for i in range(STRIPE // NB // 2):
        zr_v[pl.ds(2 * i, 2), :] = jnp.zeros((2, 16), jnp.bfloat16)
    for q in range(NB):
        pltpu.sync_copy(
            zr_v, acc_sh.at[pl.ds(s * STRIPE + q * (STRIPE // NB), STRIPE // NB)])
    pltpu.sync_copy(src_hbm.at[pl.ds(wid * CH * C, CH * C)],
                    src_v.at[pl.ds(0, CH * C)])
    pltpu.sync_copy(dst_hbm.at[pl.ds(wid * CH, CH)], dst_v.at[pl.ds(0, CH)])

    @pl.when(wid < LEFT)
    def _load_left():
        pltpu.sync_copy(src_hbm.at[pl.ds((NT * CH + wid) * C, C)],
                        src_v.at[pl.ds(CH * C, C)])
        pltpu.sync_copy(dst_hbm.at[pl.ds(NT * CH + wid, 1)],
                        dst_v.at[pl.ds(CH, 1)])

    plsc.subcore_barrier()

    def _fire(t, b):
        def body(j, carry):
            pltpu.async_copy(
                rows[b].at[pl.ds(j * C, C)],
                acc_sh.at[dst_v.at[t * KG + j]], ssems[b], add=True)
            return carry
        lax.fori_loop(0, KG, body, 0)

    def _drain(t, b):
        def body(j, carry):
            pltpu.make_async_copy(
                rows[b].at[pl.ds(j * C, C)],
                acc_sh.at[dst_v.at[t * KG + j]], ssems[b]).wait()
            return carry
        lax.fori_loop(0, KG, body, 0)

    gcps = [None, None]
    gcps[0] = pltpu.async_copy(
        g_hbm.at[src_v.at[pl.ds(0, KG * C)]], rows[0], gsems[0])
    for t in range(NG):
        b = t % 2
        nb = (t + 1) % 2
        gcps[b].wait()
        _fire(t, b)
        if t + 1 < NG:
            if t > 0:
                _drain(t - 1, nb)
            gcps[nb] = pltpu.async_copy(
                g_hbm.at[src_v.at[pl.ds((t + 1) * KG * C, KG * C)]],
                rows[nb], gsems[nb])
    _drain(NG - 2, 1)
    _drain(NG - 1, 0)

    @pl.when(wid < LEFT)
    def _do_left():
        pltpu.async_copy(
            g_hbm.at[src_v.at[pl.ds(CH * C, C)]], rows_l, gsems[0]).wait()
        pltpu.sync_copy(rows_l, acc_sh.at[dst_v.at[CH]], add=True)

    plsc.subcore_barrier()
    pltpu.sync_copy(acc_sh.at[pl.ds(s * STRIPE, STRIPE)],
                    out_hbm.at[c, pl.ds(s * STRIPE, STRIPE)])


_conv_call = pl.kernel(
    _conv_body,
    out_type=jax.ShapeDtypeStruct((2, NPAD, H), jnp.bfloat16),
    mesh=_mesh,
    compiler_params=_sc_params,
    scratch_types=[
        pltpu.VMEM(((CH + 1) * C,), jnp.int32),
        pltpu.VMEM((CH + 1, C), jnp.int32),
        pltpu.VMEM((KG * C, H), jnp.bfloat16),
        pltpu.VMEM((KG * C, H), jnp.bfloat16),
        pltpu.VMEM((C, H), jnp.bfloat16),
        pltpu.VMEM((STRIPE // NB, H), jnp.bfloat16),
        pltpu.VMEM_SHARED((NPAD, H), jnp.bfloat16),
        pltpu.SemaphoreType.DMA,
        pltpu.SemaphoreType.DMA,
        pltpu.SemaphoreType.DMA,
        pltpu.SemaphoreType.DMA,
    ],
)


def _tc1_body(x_ref, w_ref, degp_ref, g_ref, dinv_ref):
    deg = degp_ref[0, :] + degp_ref[1, :] + 1.0
    dinv = lax.rsqrt(jnp.maximum(deg, 1.0))[:N][:, None]
    h = jnp.dot(x_ref[...], w_ref[...], preferred_element_type=jnp.float32)
    g_ref[...] = (h * dinv).astype(jnp.bfloat16)
    dinv_ref[...] = dinv


def _tc2_body(acc_ref, g1_ref, dc_ref, w2_ref, b1_ref, g2_ref):
    dinv = dc_ref[...]
    ssum = (acc_ref[0, :N, :].astype(jnp.float32)
            + acc_ref[1, :N, :].astype(jnp.float32)
            + g1_ref[...].astype(jnp.float32))
    out1 = jnp.maximum(ssum * dinv + b1_ref[...], 0.0)
    h2 = jnp.dot(out1, w2_ref[...], preferred_element_type=jnp.float32)
    g2_ref[...] = (h2 * dinv).astype(jnp.bfloat16)


def _tc3_body(acc_ref, g2_ref, dc_ref, b2_ref, batch_ref, out_ref):
    dinv = dc_ref[...]
    out2 = (acc_ref[0, :N, :].astype(jnp.float32)
            + acc_ref[1, :N, :].astype(jnp.float32)
            + g2_ref[...].astype(jnp.float32)) * dinv + b2_ref[...]
    gid = lax.broadcasted_iota(jnp.int32, (G, N), 0)
    ind = jnp.where(gid == batch_ref[...], 1.0, 0.0)
    seg = jnp.dot(ind, out2, preferred_element_type=jnp.float32)
    cnt = jnp.sum(ind, axis=1, keepdims=True)
    pooled = seg[:, :CLS] / jnp.maximum(cnt, 1.0)
    m = jnp.max(pooled, axis=1, keepdims=True)
    ex = pooled - m
    lse = jnp.log(jnp.sum(jnp.exp(ex), axis=1, keepdims=True))
    out_ref[...] = ex - lse


_tc1 = pl.pallas_call(
    _tc1_body, out_shape=(jax.ShapeDtypeStruct((N, H), jnp.bfloat16),
                          jax.ShapeDtypeStruct((N, 1), jnp.float32)))
_tc2 = pl.pallas_call(
    _tc2_body, out_shape=jax.ShapeDtypeStruct((N, H), jnp.bfloat16))
_tc3 = pl.pallas_call(
    _tc3_body, out_shape=jax.ShapeDtypeStruct((G, CLS), jnp.float32))


def kernel(x, edge_index, batch, W1, b1, W2, b2):
    src = edge_index[0].astype(jnp.int32)
    dst2 = edge_index[1].astype(jnp.int32).reshape(NCH, C)

    degp = _deg_call(dst2)

    w2p = jnp.pad(W2, ((0, 0), (0, H - CLS)))
    b1r = b1.reshape(1, H)
    b2p = jnp.pad(b2, (0, H - CLS)).reshape(1, H)
    batch2 = batch.astype(jnp.int32).reshape(1, N)

    g1, dinv_col = _tc1(x, W1, degp)
    acc1 = _conv_call(g1, src, dst2)
    g2 = _tc2(acc1, g1, dinv_col, w2p, b1r)
    acc2 = _conv_call(g2, src, dst2)
    return _tc3(acc2, g2, dinv_col, b2p, batch2)
